# baseline (speedup 1.0000x reference)
import jax
import jax.numpy as jnp
from jax import lax
from jax.experimental import pallas as pl
from jax.experimental.pallas import tpu as pltpu
from jax.experimental.pallas import tpu_sc as plsc

_N_ENTRIES = 4194304
_MASK = _N_ENTRIES - 1
_P1 = 19349663
_P2 = 83492791
_GRIDF = 512.0
_B = 1048576
_F = 4
_NC = 2
_NS = 16
_NW = _NC * _NS
_NBLK = _N_ENTRIES // 128
_BLKW = _NBLK // _NW
_RB = 32
_NRCH = _BLKW // _RB
_BW = _B // _NW
_C = 512
_NCH = _BW // _C
_QB = _B // 128
_CQB = _C // 128
_NDMA = _C // 16


def _relayout_body(v3_hbm, lin_hbm, in_v, out_v, sem_in, sem_out):
    wid = lax.axis_index("s") * jnp.int32(_NC) + lax.axis_index("c")
    iota = lax.iota(jnp.int32, 16)
    e_in_vec = iota >> jnp.int32(2)
    f_vec = iota & jnp.int32(3)

    def in_slices(ci):
        b0 = wid * jnp.int32(_BLKW) + ci * jnp.int32(_RB)
        return v3_hbm.at[pl.ds(b0, _RB)], in_v.at[ci & jnp.int32(1)]

    def out_slices(ci):
        b0 = wid * jnp.int32(_BLKW) + ci * jnp.int32(_RB)
        return (out_v.at[ci & jnp.int32(1)],
                lin_hbm.at[pl.ds(b0 * jnp.int32(32), _RB * 32)])

    def fire_in(ci):
        s, d = in_slices(ci)
        pltpu.async_copy(s, d, sem_in)

    def wait_in(ci):
        s, d = in_slices(ci)
        pltpu.make_async_copy(s, d, sem_in).wait()

    def fire_out(ci):
        s, d = out_slices(ci)
        pltpu.async_copy(s, d, sem_out)

    def wait_out(ci):
        s, d = out_slices(ci)
        pltpu.make_async_copy(s, d, sem_out).wait()

    def compute(ci):
        buf = ci & jnp.int32(1)

        def t_fn(g, c2):
            for u in range(8):
                g16 = g * jnp.int32(128) + jnp.int32(u * 16)
                e = (g16 >> jnp.int32(2)) + e_in_vec
                v = plsc.load_gather(
                    in_v,
                    [jnp.zeros((16,), jnp.int32) + buf,
                     e >> jnp.int32(7), f_vec, e & jnp.int32(127)])
                out_v[buf, g * jnp.int32(8) + jnp.int32(u), pl.ds(0, 16)] = v
            return c2

        lax.fori_loop(jnp.int32(0), jnp.int32(_RB * 512 // 128), t_fn,
                      jnp.int32(0))

    fire_in(jnp.int32(0))
    fire_in(jnp.int32(1))
    wait_in(jnp.int32(0)); compute(jnp.int32(0)); fire_out(jnp.int32(0))
    fire_in(jnp.int32(2))
    wait_in(jnp.int32(1)); compute(jnp.int32(1)); fire_out(jnp.int32(1))

    def steady(ci, carry):
        fire_in(ci + jnp.int32(1))
        wait_out(ci - jnp.int32(2))
        wait_in(ci)
        compute(ci)
        fire_out(ci)
        return carry

    lax.fori_loop(jnp.int32(2), jnp.int32(_NRCH - 1), steady, jnp.int32(0))
    last = jnp.int32(_NRCH - 1)
    wait_out(last - jnp.int32(2))
    wait_in(last); compute(last); fire_out(last)
    wait_out(jnp.int32(0))
    wait_out(jnp.int32(1))


def _lookup_body(pos3_hbm, table_hbm, out3_hbm, pos_v, idx_v, lo_v, rows_v,
                 out_v, sem):
    wid = lax.axis_index("s") * jnp.int32(_NC) + lax.axis_index("c")
    iota = lax.iota(jnp.int32, 16)

    def chunk_fn(ci, carry):
        qb0 = wid * jnp.int32(_BW // 128) + ci * jnp.int32(_CQB)
        pltpu.sync_copy(pos3_hbm.at[pl.ds(qb0, _CQB)], pos_v)

        def hash_fire_fn(t, c2):
            o = t * jnp.int32(16)
            r = o >> jnp.int32(7)
            cc = o & jnp.int32(127)
            lx = (pos_v[r, 0, pl.ds(cc, 16)] * _GRIDF).astype(jnp.int32)
            ly = (pos_v[r, 1, pl.ds(cc, 16)] * _GRIDF).astype(jnp.int32)
            lz = (pos_v[r, 2, pl.ds(cc, 16)] * _GRIDF).astype(jnp.int32)
            a0 = lx
            a1 = lx + jnp.int32(1)
            b0 = ly * jnp.int32(_P1)
            b1 = b0 + jnp.int32(_P1)
            c0 = lz * jnp.int32(_P2)
            c1 = c0 + jnp.int32(_P2)
            rowv = jnp.zeros((16,), jnp.int32) + t
            colv = iota * jnp.int32(8)
            k = 0
            for bx in (a0, a1):
                for by in (b0, b1):
                    for bz in (c0, c1):
                        h = (bx ^ by ^ bz) & jnp.int32(_MASK)
                        plsc.store_scatter(
                            idx_v, [rowv, colv + jnp.int32(k)],
                            h >> jnp.int32(1))
                        plsc.store_scatter(
                            lo_v, [rowv, colv + jnp.int32(k)],
                            h & jnp.int32(1))
                        k += 1
            pltpu.async_copy(
                table_hbm.at[idx_v.at[t]],
                rows_v.at[pl.ds(t * jnp.int32(128), 128)],
                sem)
            return c2

        lax.fori_loop(jnp.int32(0), jnp.int32(_NDMA), hash_fire_fn,
                      jnp.int32(0))

        def acc_fn(t, c2):
            pltpu.make_async_copy(
                table_hbm.at[idx_v.at[t]],
                rows_v.at[pl.ds(t * jnp.int32(128), 128)],
                sem).wait()
            o = t * jnp.int32(16)
            r = o >> jnp.int32(7)
            cc = o & jnp.int32(127)
            x = pos_v[r, 0, pl.ds(cc, 16)] * _GRIDF
            y = pos_v[r, 1, pl.ds(cc, 16)] * _GRIDF
            z = pos_v[r, 2, pl.ds(cc, 16)] * _GRIDF
            fx = x - x.astype(jnp.int32).astype(jnp.float32)
            fy = y - y.astype(jnp.int32).astype(jnp.float32)
            fz = z - z.astype(jnp.int32).astype(jnp.float32)
            gx = 1.0 - fx
            gy = 1.0 - fy
            gz = 1.0 - fz
            w = []
            for cx in (fx, gx):
                for cy in (fy, gy):
                    wxy = cx * cy
                    for cz in (fz, gz):
                        w.append(wxy * cz)
            rowv = jnp.zeros((16,), jnp.int32) + t
            colv = iota * jnp.int32(8)
            base = t * jnp.int32(128)
            los = [plsc.load_gather(lo_v, [rowv, colv + jnp.int32(k)])
                   for k in range(8)]
            for f in range(_F):
                facc = None
                for k in range(8):
                    v = plsc.load_gather(
                        rows_v,
                        [base + colv + jnp.int32(k),
                         los[k] * jnp.int32(4) + jnp.int32(f)])
                    facc = w[k] * v if facc is None else facc + w[k] * v
                out_v[r, f, pl.ds(cc, 16)] = facc
            return c2

        lax.fori_loop(jnp.int32(0), jnp.int32(_NDMA), acc_fn, jnp.int32(0))
        pltpu.sync_copy(out_v, out3_hbm.at[pl.ds(qb0, _CQB)])
        return carry

    lax.fori_loop(jnp.int32(0), jnp.int32(_NCH), chunk_fn, jnp.int32(0))


@jax.jit
def kernel(position, hash_table):
    table_v3 = hash_table.T.reshape(4, _NBLK, 128).transpose(1, 0, 2)
    pos4 = jnp.pad(position.astype(jnp.float32), ((0, 0), (0, 1)))
    pos_v3 = pos4.T.reshape(4, _QB, 128).transpose(1, 0, 2)

    relayout = pl.kernel(
        _relayout_body,
        compiler_params=pltpu.CompilerParams(use_tc_tiling_on_sc=False,
                                             needs_layout_passes=False),
        out_type=jax.ShapeDtypeStruct((_N_ENTRIES // 4, 16), jnp.float32),
        mesh=plsc.VectorSubcoreMesh(core_axis_name="c", subcore_axis_name="s"),
        scratch_types=[
            pltpu.VMEM((2, _RB, 4, 128), jnp.float32),
            pltpu.VMEM((2, _RB * 32, 16), jnp.float32),
            pltpu.SemaphoreType.DMA,
            pltpu.SemaphoreType.DMA,
        ],
    )
    table_lin = relayout(table_v3)

    lookup = pl.kernel(
        _lookup_body,
        compiler_params=pltpu.CompilerParams(use_tc_tiling_on_sc=False,
                                             needs_layout_passes=False),
        out_type=jax.ShapeDtypeStruct((_QB, 4, 128), jnp.float32),
        mesh=plsc.VectorSubcoreMesh(core_axis_name="c", subcore_axis_name="s"),
        scratch_types=[
            pltpu.VMEM((_CQB, 4, 128), jnp.float32),
            pltpu.VMEM((_NDMA, 128), jnp.int32),
            pltpu.VMEM((_NDMA, 128), jnp.int32),
            pltpu.VMEM((8 * _C, 8), jnp.float32),
            pltpu.VMEM((_CQB, 4, 128), jnp.float32),
            pltpu.SemaphoreType.DMA,
        ],
    )
    out3 = lookup(pos_v3, table_lin.reshape(_N_ENTRIES // 2, 8))
    return out3.transpose(1, 0, 2).reshape(4, _B).T


# speedup vs baseline: 1.4527x; 1.4527x over previous
import jax
import jax.numpy as jnp
from jax import lax
from jax.experimental import pallas as pl
from jax.experimental.pallas import tpu as pltpu
from jax.experimental.pallas import tpu_sc as plsc

_N_ENTRIES = 4194304
_MASK = _N_ENTRIES - 1
_P1 = 19349663
_P2 = 83492791
_GRIDF = 512.0
_B = 1048576
_F = 4
_NC = 2
_NS = 16
_NW = _NC * _NS
_NBLK = _N_ENTRIES // 128
_BLKW = _NBLK // _NW
_RB = 32
_NRCH = _BLKW // _RB
_BW = _B // _NW
_C = 256
_NCH = _BW // _C
_QB = _B // 128
_CQB = _C // 128
_NDMA = _C // 16


def _relayout_body(v3_hbm, lin_hbm, in_v, out_v, sem_in, sem_out):
    wid = lax.axis_index("s") * jnp.int32(_NC) + lax.axis_index("c")
    iota = lax.iota(jnp.int32, 16)
    e_in_vec = iota >> jnp.int32(2)
    f_vec = iota & jnp.int32(3)

    def in_slices(ci):
        b0 = wid * jnp.int32(_BLKW) + ci * jnp.int32(_RB)
        return v3_hbm.at[pl.ds(b0, _RB)], in_v.at[ci & jnp.int32(1)]

    def out_slices(ci):
        b0 = wid * jnp.int32(_BLKW) + ci * jnp.int32(_RB)
        return (out_v.at[ci & jnp.int32(1)],
                lin_hbm.at[pl.ds(b0 * jnp.int32(32), _RB * 32)])

    def fire_in(ci):
        s, d = in_slices(ci)
        pltpu.async_copy(s, d, sem_in)

    def wait_in(ci):
        s, d = in_slices(ci)
        pltpu.make_async_copy(s, d, sem_in).wait()

    def fire_out(ci):
        s, d = out_slices(ci)
        pltpu.async_copy(s, d, sem_out)

    def wait_out(ci):
        s, d = out_slices(ci)
        pltpu.make_async_copy(s, d, sem_out).wait()

    def compute(ci):
        buf = ci & jnp.int32(1)

        def t_fn(g, c2):
            for u in range(8):
                g16 = g * jnp.int32(128) + jnp.int32(u * 16)
                e = (g16 >> jnp.int32(2)) + e_in_vec
                v = plsc.load_gather(
                    in_v,
                    [jnp.zeros((16,), jnp.int32) + buf,
                     e >> jnp.int32(7), f_vec, e & jnp.int32(127)])
                out_v[buf, g * jnp.int32(8) + jnp.int32(u), pl.ds(0, 16)] = v
            return c2

        lax.fori_loop(jnp.int32(0), jnp.int32(_RB * 512 // 128), t_fn,
                      jnp.int32(0))

    fire_in(jnp.int32(0))
    fire_in(jnp.int32(1))
    wait_in(jnp.int32(0)); compute(jnp.int32(0)); fire_out(jnp.int32(0))
    fire_in(jnp.int32(2))
    wait_in(jnp.int32(1)); compute(jnp.int32(1)); fire_out(jnp.int32(1))

    def steady(ci, carry):
        fire_in(ci + jnp.int32(1))
        wait_out(ci - jnp.int32(2))
        wait_in(ci)
        compute(ci)
        fire_out(ci)
        return carry

    lax.fori_loop(jnp.int32(2), jnp.int32(_NRCH - 1), steady, jnp.int32(0))
    last = jnp.int32(_NRCH - 1)
    wait_out(last - jnp.int32(2))
    wait_in(last); compute(last); fire_out(last)
    wait_out(jnp.int32(0))
    wait_out(jnp.int32(1))


def _lookup_body(pos3_hbm, table_hbm, out3_hbm, pos_v, idx_v, lo_v, rows_v,
                 out_v, sem_pos, sem_g):
    wid = lax.axis_index("s") * jnp.int32(_NC) + lax.axis_index("c")
    iota = lax.iota(jnp.int32, 16)

    def qb0_of(ci):
        return wid * jnp.int32(_BW // 128) + ci * jnp.int32(_CQB)

    def fire_pos(ci):
        pltpu.async_copy(pos3_hbm.at[pl.ds(qb0_of(ci), _CQB)],
                         pos_v.at[ci & jnp.int32(3)], sem_pos)

    def wait_pos(ci):
        pltpu.make_async_copy(pos3_hbm.at[pl.ds(qb0_of(ci), _CQB)],
                              pos_v.at[ci & jnp.int32(3)], sem_pos).wait()

    def gather_slices(ci, t):
        buf = ci & jnp.int32(1)
        return (table_hbm.at[idx_v.at[buf, t]],
                rows_v.at[buf, pl.ds(t * jnp.int32(128), 128)])

    def hash_fire(ci):
        pbuf = ci & jnp.int32(3)
        buf = ci & jnp.int32(1)

        def hash_fire_fn(t, c2):
            o = t * jnp.int32(16)
            r = o >> jnp.int32(7)
            cc = o & jnp.int32(127)
            lx = (pos_v[pbuf, r, 0, pl.ds(cc, 16)] * _GRIDF).astype(jnp.int32)
            ly = (pos_v[pbuf, r, 1, pl.ds(cc, 16)] * _GRIDF).astype(jnp.int32)
            lz = (pos_v[pbuf, r, 2, pl.ds(cc, 16)] * _GRIDF).astype(jnp.int32)
            a0 = lx
            a1 = lx + jnp.int32(1)
            b0 = ly * jnp.int32(_P1)
            b1 = b0 + jnp.int32(_P1)
            c0 = lz * jnp.int32(_P2)
            c1 = c0 + jnp.int32(_P2)
            rowv = jnp.zeros((16,), jnp.int32) + t
            colv = iota * jnp.int32(8)
            k = 0
            for bx in (a0, a1):
                for by in (b0, b1):
                    for bz in (c0, c1):
                        h = (bx ^ by ^ bz) & jnp.int32(_MASK)
                        plsc.store_scatter(
                            idx_v, [jnp.zeros((16,), jnp.int32) + buf, rowv,
                                    colv + jnp.int32(k)],
                            h >> jnp.int32(2))
                        plsc.store_scatter(
                            lo_v, [jnp.zeros((16,), jnp.int32) + buf, rowv,
                                   colv + jnp.int32(k)],
                            h & jnp.int32(3))
                        k += 1
            s, d = gather_slices(ci, t)
            pltpu.async_copy(s, d, sem_g)
            return c2

        lax.fori_loop(jnp.int32(0), jnp.int32(_NDMA), hash_fire_fn,
                      jnp.int32(0))

    def drain_acc(ci):
        pbuf = ci & jnp.int32(3)
        buf = ci & jnp.int32(1)

        def acc_fn(t, c2):
            s, d = gather_slices(ci, t)
            pltpu.make_async_copy(s, d, sem_g).wait()
            o = t * jnp.int32(16)
            r = o >> jnp.int32(7)
            cc = o & jnp.int32(127)
            x = pos_v[pbuf, r, 0, pl.ds(cc, 16)] * _GRIDF
            y = pos_v[pbuf, r, 1, pl.ds(cc, 16)] * _GRIDF
            z = pos_v[pbuf, r, 2, pl.ds(cc, 16)] * _GRIDF
            fx = x - x.astype(jnp.int32).astype(jnp.float32)
            fy = y - y.astype(jnp.int32).astype(jnp.float32)
            fz = z - z.astype(jnp.int32).astype(jnp.float32)
            gx = 1.0 - fx
            gy = 1.0 - fy
            gz = 1.0 - fz
            w = []
            for cx in (fx, gx):
                for cy in (fy, gy):
                    wxy = cx * cy
                    for cz in (fz, gz):
                        w.append(wxy * cz)
            rowv = jnp.zeros((16,), jnp.int32) + t
            colv = iota * jnp.int32(8)
            base = t * jnp.int32(128)
            bufv = jnp.zeros((16,), jnp.int32) + buf
            los = [plsc.load_gather(lo_v, [bufv, rowv, colv + jnp.int32(k)])
                   for k in range(8)]
            for f in range(_F):
                facc = None
                for k in range(8):
                    v = plsc.load_gather(
                        rows_v,
                        [bufv, base + colv + jnp.int32(k),
                         los[k] * jnp.int32(4) + jnp.int32(f)])
                    facc = w[k] * v if facc is None else facc + w[k] * v
                out_v[r, f, pl.ds(cc, 16)] = facc
            return c2

        lax.fori_loop(jnp.int32(0), jnp.int32(_NDMA), acc_fn, jnp.int32(0))
        pltpu.sync_copy(out_v, out3_hbm.at[pl.ds(qb0_of(ci), _CQB)])

    fire_pos(jnp.int32(0))
    fire_pos(jnp.int32(1))
    wait_pos(jnp.int32(0))
    hash_fire(jnp.int32(0))

    def steady(ci, carry):
        wait_pos(ci)
        hash_fire(ci)
        fire_pos((ci + jnp.int32(1)) & jnp.int32(_NCH - 1))
        drain_acc(ci - jnp.int32(1))
        return carry

    lax.fori_loop(jnp.int32(1), jnp.int32(_NCH), steady, jnp.int32(0))
    drain_acc(jnp.int32(_NCH - 1))
    wait_pos(jnp.int32(0))


@jax.jit
def kernel(position, hash_table):
    table_v3 = hash_table.T.reshape(4, _NBLK, 128).transpose(1, 0, 2)
    pos4 = jnp.pad(position.astype(jnp.float32), ((0, 0), (0, 1)))
    pos_v3 = pos4.T.reshape(4, _QB, 128).transpose(1, 0, 2)

    relayout = pl.kernel(
        _relayout_body,
        compiler_params=pltpu.CompilerParams(use_tc_tiling_on_sc=False,
                                             needs_layout_passes=False),
        out_type=jax.ShapeDtypeStruct((_N_ENTRIES // 4, 16), jnp.float32),
        mesh=plsc.VectorSubcoreMesh(core_axis_name="c", subcore_axis_name="s"),
        scratch_types=[
            pltpu.VMEM((2, _RB, 4, 128), jnp.float32),
            pltpu.VMEM((2, _RB * 32, 16), jnp.float32),
            pltpu.SemaphoreType.DMA,
            pltpu.SemaphoreType.DMA,
        ],
    )
    table_lin = relayout(table_v3)

    lookup = pl.kernel(
        _lookup_body,
        compiler_params=pltpu.CompilerParams(use_tc_tiling_on_sc=False,
                                             needs_layout_passes=False),
        out_type=jax.ShapeDtypeStruct((_QB, 4, 128), jnp.float32),
        mesh=plsc.VectorSubcoreMesh(core_axis_name="c", subcore_axis_name="s"),
        scratch_types=[
            pltpu.VMEM((4, _CQB, 4, 128), jnp.float32),
            pltpu.VMEM((2, _NDMA, 128), jnp.int32),
            pltpu.VMEM((2, _NDMA, 128), jnp.int32),
            pltpu.VMEM((2, 8 * _C, 16), jnp.float32),
            pltpu.VMEM((_CQB, 4, 128), jnp.float32),
            pltpu.SemaphoreType.DMA,
            pltpu.SemaphoreType.DMA,
        ],
    )
    out3 = lookup(pos_v3, table_lin)
    return out3.transpose(1, 0, 2).reshape(4, _B).T
